# baseline (speedup 1.0000x reference)
import functools

import jax
import jax.numpy as jnp
from jax import lax
from jax.experimental import pallas as pl
from jax.experimental.pallas import tpu as pltpu
from jax.experimental.pallas import tpu_sc as plsc

_N_PATCHES = 1024
_HIDDEN = 768
_NC = 2
_NS = 16
_NW = _NC * _NS
_ROWS_PER_W = _N_PATCHES // _NW


_FUSE = 4
_SUP_D = _FUSE * _HIDDEN
_SUP_N = _N_PATCHES // _FUSE
_SUP_PER_W = _SUP_N // _NW
_HALF = _SUP_PER_W // 2


@functools.cache
def _sc_lookup():
    mesh = plsc.VectorSubcoreMesh(core_axis_name="c", subcore_axis_name="s")

    @functools.partial(
        pl.kernel,
        out_type=jax.ShapeDtypeStruct((_SUP_N, _SUP_D), jnp.float32),
        mesh=mesh,
        scratch_types=[
            pltpu.VMEM((2, _HALF), jnp.int32),
            pltpu.VMEM((_HALF, _SUP_D), jnp.float32),
            pltpu.VMEM((_HALF, _SUP_D), jnp.float32),
            pltpu.SemaphoreType.DMA,
            pltpu.SemaphoreType.DMA,
            pltpu.SemaphoreType.DMA,
        ],
    )
    def body(table_hbm, idx_hbm, out_hbm, idx_v, rows_a, rows_b, sem_a, sem_b,
             sem_st):
        wid = lax.axis_index("s") * _NC + lax.axis_index("c")
        base = wid * _SUP_PER_W
        pltpu.sync_copy(idx_hbm.at[wid], idx_v)
        ga = pltpu.async_copy(table_hbm.at[idx_v.at[0]], rows_a, sem_a)
        gb = pltpu.async_copy(table_hbm.at[idx_v.at[1]], rows_b, sem_b)
        ga.wait()
        st = pltpu.async_copy(rows_a, out_hbm.at[pl.ds(base, _HALF)], sem_st)
        gb.wait()
        pltpu.sync_copy(rows_b, out_hbm.at[pl.ds(base + _HALF, _HALF)])
        st.wait()

    return body


def kernel(x, lev, table):
    lev32 = jnp.asarray(lev, jnp.int32)
    idx = (lev32 - 1) * _SUP_N + jnp.arange(_SUP_N, dtype=jnp.int32)
    out = _sc_lookup()(table.reshape(-1, _SUP_D), idx.reshape(_NW, 2, _HALF))
    return out.reshape(_N_PATCHES, _HIDDEN)[None, : x.shape[1]]


# speedup vs baseline: 2.4143x; 2.4143x over previous
import functools

import jax
import jax.numpy as jnp
from jax import lax
from jax.experimental import pallas as pl
from jax.experimental.pallas import tpu as pltpu
from jax.experimental.pallas import tpu_sc as plsc

_N_PATCHES = 1024
_HIDDEN = 768
_NC = 2
_NS = 16
_NW = _NC * _NS
_BLOCK = _N_PATCHES // _NW


@functools.cache
def _sc_lookup():
    mesh = plsc.VectorSubcoreMesh(core_axis_name="c", subcore_axis_name="s")

    @functools.partial(
        pl.kernel,
        out_type=jax.ShapeDtypeStruct((_NW, _BLOCK, _HIDDEN), jnp.float32),
        mesh=mesh,
        scratch_types=[
            pltpu.VMEM((1, 1), jnp.int32),
            pltpu.VMEM((1, _BLOCK, _HIDDEN), jnp.float32),
            pltpu.SemaphoreType.DMA,
        ],
    )
    def body(table_hbm, idx_hbm, out_hbm, idx_v, rows_v, sem):
        wid = lax.axis_index("s") * _NC + lax.axis_index("c")
        pltpu.sync_copy(idx_hbm.at[wid], idx_v)
        pltpu.async_copy(table_hbm.at[idx_v.at[0]], rows_v, sem).wait()
        pltpu.sync_copy(rows_v, out_hbm.at[pl.ds(wid, 1)])

    return body


def kernel(x, lev, table):
    lev32 = jnp.asarray(lev, jnp.int32)
    idx = (lev32 - 1) * _NW + jnp.arange(_NW, dtype=jnp.int32)
    out = _sc_lookup()(
        table.reshape(-1, _BLOCK, _HIDDEN), idx.reshape(_NW, 1, 1))
    return out.reshape(1, _N_PATCHES, _HIDDEN)[:, : x.shape[1]]
